# baseline (speedup 1.0000x reference)
import functools

import jax
import jax.numpy as jnp
from jax import lax
from jax.experimental import pallas as pl
from jax.experimental.pallas import tpu as pltpu
from jax.experimental.pallas import tpu_sc as plsc

B, D, S = 4, 1024, 2048
K = 256
_ROWS_PER_STEP = 512
_G, _L = 16, 128

_BTC = 2
_BSC = B - _BTC
_NC, _NS, _SL = 2, 16, 16
_NW = _NC * _NS
_WPB = _NW // _BSC
_ROWS_PER_W = D // _WPB
_CHUNK = 32
_NCHUNK = _ROWS_PER_W // _CHUNK


def _mean_kernel(w_ref, out_ref, acc_ref):
    j = pl.program_id(1)
    nj = pl.num_programs(1)

    @pl.when(j == 0)
    def _init():
        acc_ref[...] = jnp.zeros_like(acc_ref)

    acc = acc_ref[...]
    for g in range(_ROWS_PER_STEP // 8):
        acc = acc + w_ref[0, 8 * g:8 * g + 8, :]
    acc_ref[...] = acc

    @pl.when(j == nj - 1)
    def _finish():
        a = acc_ref[...]
        t = a[0:4] + a[4:8]
        u = t[0:2] + t[2:4]
        s = u[0:1] + u[1:2]
        out_ref[0] = s * (1.0 / S)


def _topk_kernel(m_ref, idx_ref):
    v = m_ref[...].reshape(B, _G, _L)
    jg = lax.broadcasted_iota(jnp.int32, (B, _G, _L), 1)
    jl = lax.broadcasted_iota(jnp.int32, (B, _G, _L), 2)
    i = jg * _L + jl

    def cmpex(v, i, d, k):
        if d >= _L:
            dd = d // _L
            is_lo = (jg & dd) == 0
            vp = jnp.where(is_lo, jnp.roll(v, -dd, axis=1),
                           jnp.roll(v, dd, axis=1))
            ip = jnp.where(is_lo, jnp.roll(i, -dd, axis=1),
                           jnp.roll(i, dd, axis=1))
        else:
            is_lo = (jl & d) == 0
            vp = jnp.where(is_lo, jnp.roll(v, -d, axis=2),
                           jnp.roll(v, d, axis=2))
            ip = jnp.where(is_lo, jnp.roll(i, -d, axis=2),
                           jnp.roll(i, d, axis=2))
        if k >= S:
            asc = jnp.full(v.shape, True)
        elif k >= _L:
            asc = (jg & (k // _L)) == 0
        else:
            asc = (jl & k) == 0
        own_first = (v > vp) | ((v == vp) & (i < ip))
        take_own = own_first == (is_lo == asc)
        return jnp.where(take_own, v, vp), jnp.where(take_own, i, ip)

    k = 2
    while k <= S:
        d = k // 2
        while d >= 1:
            v, i = cmpex(v, i, d, k)
            d //= 2
        k *= 2

    idx_ref[...] = i[:, 0:K // _L, :].reshape(B, K)


def _gather_sc(x_hbm, idx_hbm, out_hbm, idx_v, row_v, orow_v, sem):
    wid = lax.axis_index("s") * _NC + lax.axis_index("c")
    b = wid // _WPB
    r0 = (wid % _WPB) * _ROWS_PER_W
    pltpu.sync_copy(idx_hbm.at[b], idx_v)

    def chunk_body(c, carry):
        base = r0 + c * _CHUNK
        pltpu.sync_copy(x_hbm.at[b, pl.ds(base, _CHUNK)], row_v)
        for i in range(_CHUNK):
            ri = jnp.full((_SL,), i, jnp.int32)
            for j in range(K // _SL):
                ids = idx_v[pl.ds(j * _SL, _SL)]
                orow_v[i, pl.ds(j * _SL, _SL)] = plsc.load_gather(
                    row_v, [ri, ids])
        pltpu.sync_copy(orow_v, out_hbm.at[b, pl.ds(base, _CHUNK)])
        return carry

    lax.fori_loop(0, _NCHUNK, chunk_body, 0)


_gather_call = pl.kernel(
    _gather_sc,
    mesh=plsc.VectorSubcoreMesh(core_axis_name="c", subcore_axis_name="s"),
    out_type=jax.ShapeDtypeStruct((_BSC, D, K), jnp.float32),
    scratch_types=[
        pltpu.VMEM((K,), jnp.int32),
        pltpu.VMEM((_CHUNK, S), jnp.float32),
        pltpu.VMEM((_CHUNK, K), jnp.float32),
        pltpu.SemaphoreType.DMA,
    ],
    compiler_params=pltpu.CompilerParams(needs_layout_passes=False),
)


def _gather_tc(x_ref, idx_ref, out_ref):
    idx_row = idx_ref[0]
    onehot = (lax.broadcasted_iota(jnp.int32, (S, K), 0)
              == idx_row).astype(jnp.float32)
    out_ref[0] = jnp.dot(x_ref[0], onehot,
                         preferred_element_type=jnp.float32,
                         precision=lax.Precision.HIGHEST)


@functools.partial(jax.jit)
def kernel(x, w):
    nsteps = S // _ROWS_PER_STEP
    w_mean = pl.pallas_call(
        _mean_kernel,
        grid=(B, nsteps),
        in_specs=[pl.BlockSpec((1, _ROWS_PER_STEP, S),
                               lambda b, j: (b, j, 0))],
        out_specs=pl.BlockSpec((1, 1, S), lambda b, j: (b, 0, 0)),
        out_shape=jax.ShapeDtypeStruct((B, 1, S), jnp.float32),
        scratch_shapes=[pltpu.VMEM((8, S), jnp.float32)],
        compiler_params=pltpu.CompilerParams(
            dimension_semantics=("arbitrary", "arbitrary")),
    )(w)

    idx = pl.pallas_call(
        _topk_kernel,
        out_shape=jax.ShapeDtypeStruct((B, K), jnp.int32),
    )(w_mean.reshape(B, S))

    out_sc = _gather_call(lax.slice_in_dim(x, _BTC, B, axis=0),
                          lax.slice_in_dim(idx, _BTC, B, axis=0))

    idx3 = idx.reshape(B, 1, K)
    out_tc = pl.pallas_call(
        _gather_tc,
        grid=(_BTC,),
        in_specs=[
            pl.BlockSpec((1, D, S), lambda b: (b, 0, 0)),
            pl.BlockSpec((1, 1, K), lambda b: (b, 0, 0)),
        ],
        out_specs=pl.BlockSpec((1, D, K), lambda b: (b, 0, 0)),
        out_shape=jax.ShapeDtypeStruct((_BTC, D, K), jnp.float32),
        compiler_params=pltpu.CompilerParams(
            dimension_semantics=("arbitrary",)),
    )(x, idx3)
    return jnp.concatenate([out_tc, out_sc], axis=0)


# speedup vs baseline: 1.1956x; 1.1956x over previous
import functools

import jax
import jax.numpy as jnp
from jax import lax
from jax.experimental import pallas as pl
from jax.experimental.pallas import tpu as pltpu
from jax.experimental.pallas import tpu_sc as plsc

B, D, S = 4, 1024, 2048
K = 256
_ROWS_PER_STEP = 512
_G, _L = 16, 128

_BTC = 2
_BSC = B - _BTC
_NC, _NS, _SL = 2, 16, 16
_NW = _NC * _NS
_WPB = _NW // _BSC
_ROWS_PER_W = D // _WPB
_CHUNK = 32
_NCHUNK = _ROWS_PER_W // _CHUNK


def _mean_kernel(w_ref, out_ref, acc_ref):
    j = pl.program_id(1)
    nj = pl.num_programs(1)

    @pl.when(j == 0)
    def _init():
        acc_ref[...] = jnp.zeros_like(acc_ref)

    acc = acc_ref[...]
    for g in range(_ROWS_PER_STEP // 8):
        acc = acc + w_ref[0, 8 * g:8 * g + 8, :]
    acc_ref[...] = acc

    @pl.when(j == nj - 1)
    def _finish():
        a = acc_ref[...]
        t = a[0:4] + a[4:8]
        u = t[0:2] + t[2:4]
        s = u[0:1] + u[1:2]
        out_ref[0] = s * (1.0 / S)


def _topk_kernel(m_ref, idx_ref):
    v = m_ref[...].reshape(B, _G, _L)
    jg = lax.broadcasted_iota(jnp.int32, (B, _G, _L), 1)
    jl = lax.broadcasted_iota(jnp.int32, (B, _G, _L), 2)
    i = jg * _L + jl

    def cmpex(v, i, d, k):
        if d >= _L:
            dd = d // _L
            is_lo = (jg & dd) == 0
            vp = jnp.where(is_lo, jnp.roll(v, -dd, axis=1),
                           jnp.roll(v, dd, axis=1))
            ip = jnp.where(is_lo, jnp.roll(i, -dd, axis=1),
                           jnp.roll(i, dd, axis=1))
        else:
            is_lo = (jl & d) == 0
            vp = jnp.where(is_lo, jnp.roll(v, -d, axis=2),
                           jnp.roll(v, d, axis=2))
            ip = jnp.where(is_lo, jnp.roll(i, -d, axis=2),
                           jnp.roll(i, d, axis=2))
        if k >= S:
            asc = jnp.full(v.shape, True)
        elif k >= _L:
            asc = (jg & (k // _L)) == 0
        else:
            asc = (jl & k) == 0
        own_first = (v > vp) | ((v == vp) & (i < ip))
        take_own = own_first == (is_lo == asc)
        return jnp.where(take_own, v, vp), jnp.where(take_own, i, ip)

    k = 2
    while k <= S:
        d = k // 2
        while d >= 1:
            v, i = cmpex(v, i, d, k)
            d //= 2
        k *= 2

    idx_ref[...] = i[:, 0:K // _L, :].reshape(B, K)


def _gather_sc(x_hbm, idx_hbm, out_hbm, idx_v, row_v, orow_v, sem):
    wid = lax.axis_index("s") * _NC + lax.axis_index("c")
    bo = wid // _WPB
    b = bo + _BTC
    r0 = (wid % _WPB) * _ROWS_PER_W
    pltpu.sync_copy(idx_hbm.at[b], idx_v)

    def chunk_body(c, carry):
        base = r0 + c * _CHUNK
        pltpu.sync_copy(x_hbm.at[b, pl.ds(base, _CHUNK)], row_v)
        for i in range(_CHUNK):
            ri = jnp.full((_SL,), i, jnp.int32)
            for j in range(K // _SL):
                ids = idx_v[pl.ds(j * _SL, _SL)]
                orow_v[i, pl.ds(j * _SL, _SL)] = plsc.load_gather(
                    row_v, [ri, ids])
        pltpu.sync_copy(orow_v, out_hbm.at[bo, pl.ds(base, _CHUNK)])
        return carry

    lax.fori_loop(0, _NCHUNK, chunk_body, 0)


_gather_call = pl.kernel(
    _gather_sc,
    mesh=plsc.VectorSubcoreMesh(core_axis_name="c", subcore_axis_name="s"),
    out_type=jax.ShapeDtypeStruct((_BSC, D, K), jnp.float32),
    scratch_types=[
        pltpu.VMEM((K,), jnp.int32),
        pltpu.VMEM((_CHUNK, S), jnp.float32),
        pltpu.VMEM((_CHUNK, K), jnp.float32),
        pltpu.SemaphoreType.DMA,
    ],
    compiler_params=pltpu.CompilerParams(needs_layout_passes=False),
)


def _gather_tc(x_ref, idx_ref, out_ref):
    idx_row = idx_ref[0]
    onehot = (lax.broadcasted_iota(jnp.int32, (S, K), 0)
              == idx_row).astype(jnp.float32)
    out_ref[0] = jnp.dot(x_ref[0], onehot,
                         preferred_element_type=jnp.float32,
                         precision=lax.Precision.HIGHEST)


@functools.partial(jax.jit)
def kernel(x, w):
    nsteps = S // _ROWS_PER_STEP
    w_mean = pl.pallas_call(
        _mean_kernel,
        grid=(B, nsteps),
        in_specs=[pl.BlockSpec((1, _ROWS_PER_STEP, S),
                               lambda b, j: (b, j, 0))],
        out_specs=pl.BlockSpec((1, 1, S), lambda b, j: (b, 0, 0)),
        out_shape=jax.ShapeDtypeStruct((B, 1, S), jnp.float32),
        scratch_shapes=[pltpu.VMEM((8, S), jnp.float32)],
        compiler_params=pltpu.CompilerParams(
            dimension_semantics=("arbitrary", "arbitrary")),
    )(w)

    idx = pl.pallas_call(
        _topk_kernel,
        out_shape=jax.ShapeDtypeStruct((B, K), jnp.int32),
    )(w_mean.reshape(B, S))

    out_sc = _gather_call(x, idx)

    idx3 = idx.reshape(B, 1, K)
    out_tc = pl.pallas_call(
        _gather_tc,
        grid=(_BTC,),
        in_specs=[
            pl.BlockSpec((1, D, S), lambda b: (b, 0, 0)),
            pl.BlockSpec((1, 1, K), lambda b: (b, 0, 0)),
        ],
        out_specs=pl.BlockSpec((1, D, K), lambda b: (b, 0, 0)),
        out_shape=jax.ShapeDtypeStruct((_BTC, D, K), jnp.float32),
        compiler_params=pltpu.CompilerParams(
            dimension_semantics=("arbitrary",)),
    )(x, idx3)
    return jnp.concatenate([out_tc, out_sc], axis=0)


# speedup vs baseline: 1.2791x; 1.0698x over previous
import functools

import jax
import jax.numpy as jnp
from jax import lax
from jax.experimental import pallas as pl
from jax.experimental.pallas import tpu as pltpu
from jax.experimental.pallas import tpu_sc as plsc

B, D, S = 4, 1024, 2048
K = 256
_ROWS_PER_STEP = 1024
_G, _L = 16, 128

_BTC = 3
_BSC = B - _BTC
_NC, _NS, _SL = 2, 16, 16
_NW = _NC * _NS
_WPB = _NW // _BSC
_ROWS_PER_W = D // _WPB
_CHUNK = 32
_NCHUNK = _ROWS_PER_W // _CHUNK


def _mean_kernel(w_ref, out_ref, acc_ref):
    j = pl.program_id(1)
    nj = pl.num_programs(1)

    @pl.when(j == 0)
    def _init():
        acc_ref[...] = jnp.zeros_like(acc_ref)

    acc = acc_ref[...]
    for g in range(_ROWS_PER_STEP // 8):
        acc = acc + w_ref[0, 8 * g:8 * g + 8, :]
    acc_ref[...] = acc

    @pl.when(j == nj - 1)
    def _finish():
        a = acc_ref[...]
        t = a[0:4] + a[4:8]
        u = t[0:2] + t[2:4]
        s = u[0:1] + u[1:2]
        out_ref[0] = s * (1.0 / S)


def _topk_kernel(m_ref, idx_ref):
    v = m_ref[...].reshape(B, _G, _L)
    jg = lax.broadcasted_iota(jnp.int32, (B, _G, _L), 1)
    jl = lax.broadcasted_iota(jnp.int32, (B, _G, _L), 2)
    i = jg * _L + jl

    def cmpex(v, i, d, k):
        if d >= _L:
            dd = d // _L
            is_lo = (jg & dd) == 0
            vp = jnp.where(is_lo, jnp.roll(v, -dd, axis=1),
                           jnp.roll(v, dd, axis=1))
            ip = jnp.where(is_lo, jnp.roll(i, -dd, axis=1),
                           jnp.roll(i, dd, axis=1))
        else:
            is_lo = (jl & d) == 0
            vp = jnp.where(is_lo, jnp.roll(v, -d, axis=2),
                           jnp.roll(v, d, axis=2))
            ip = jnp.where(is_lo, jnp.roll(i, -d, axis=2),
                           jnp.roll(i, d, axis=2))
        if k >= S:
            asc = jnp.full(v.shape, True)
        elif k >= _L:
            asc = (jg & (k // _L)) == 0
        else:
            asc = (jl & k) == 0
        own_first = (v > vp) | ((v == vp) & (i < ip))
        take_own = own_first == (is_lo == asc)
        return jnp.where(take_own, v, vp), jnp.where(take_own, i, ip)

    k = 2
    while k <= S:
        d = k // 2
        while d >= 1:
            v, i = cmpex(v, i, d, k)
            d //= 2
        k *= 2

    idx_ref[...] = i[:, 0:K // _L, :].reshape(B, K)


def _gather_sc(x_hbm, idx_hbm, out_hbm, idx_v, row_v, orow_v, sem):
    wid = lax.axis_index("s") * _NC + lax.axis_index("c")
    bo = wid // _WPB
    b = bo + _BTC
    r0 = (wid % _WPB) * _ROWS_PER_W
    pltpu.sync_copy(idx_hbm.at[b], idx_v)

    def chunk_body(c, carry):
        base = r0 + c * _CHUNK
        pltpu.sync_copy(x_hbm.at[b, pl.ds(base, _CHUNK)], row_v)
        for i in range(_CHUNK):
            ri = jnp.full((_SL,), i, jnp.int32)
            for j in range(K // _SL):
                ids = idx_v[pl.ds(j * _SL, _SL)]
                orow_v[i, pl.ds(j * _SL, _SL)] = plsc.load_gather(
                    row_v, [ri, ids])
        pltpu.sync_copy(orow_v, out_hbm.at[bo, pl.ds(base, _CHUNK)])
        return carry

    lax.fori_loop(0, _NCHUNK, chunk_body, 0)


_gather_call = pl.kernel(
    _gather_sc,
    mesh=plsc.VectorSubcoreMesh(core_axis_name="c", subcore_axis_name="s"),
    out_type=jax.ShapeDtypeStruct((_BSC, D, K), jnp.float32),
    scratch_types=[
        pltpu.VMEM((K,), jnp.int32),
        pltpu.VMEM((_CHUNK, S), jnp.float32),
        pltpu.VMEM((_CHUNK, K), jnp.float32),
        pltpu.SemaphoreType.DMA,
    ],
    compiler_params=pltpu.CompilerParams(needs_layout_passes=False),
)


def _gather_tc(x_ref, idx_ref, out_ref):
    idx_row = idx_ref[0]
    onehot = (lax.broadcasted_iota(jnp.int32, (S, K), 0)
              == idx_row).astype(jnp.bfloat16)
    xb = x_ref[0]
    mask = jnp.uint32(0xFFFF0000)
    hi_f = lax.bitcast_convert_type(
        lax.bitcast_convert_type(xb, jnp.uint32) & mask, jnp.float32)
    r = xb - hi_f
    mid_f = lax.bitcast_convert_type(
        lax.bitcast_convert_type(r, jnp.uint32) & mask, jnp.float32)
    lo_f = r - mid_f
    acc = jnp.dot(hi_f.astype(jnp.bfloat16), onehot,
                  preferred_element_type=jnp.float32)
    acc = acc + jnp.dot(mid_f.astype(jnp.bfloat16), onehot,
                        preferred_element_type=jnp.float32)
    acc = acc + jnp.dot(lo_f.astype(jnp.bfloat16), onehot,
                        preferred_element_type=jnp.float32)
    out_ref[0] = acc


@functools.partial(jax.jit)
def kernel(x, w):
    nsteps = S // _ROWS_PER_STEP
    w_mean = pl.pallas_call(
        _mean_kernel,
        grid=(B, nsteps),
        in_specs=[pl.BlockSpec((1, _ROWS_PER_STEP, S),
                               lambda b, j: (b, j, 0))],
        out_specs=pl.BlockSpec((1, 1, S), lambda b, j: (b, 0, 0)),
        out_shape=jax.ShapeDtypeStruct((B, 1, S), jnp.float32),
        scratch_shapes=[pltpu.VMEM((8, S), jnp.float32)],
        compiler_params=pltpu.CompilerParams(
            dimension_semantics=("arbitrary", "arbitrary")),
    )(w)

    idx = pl.pallas_call(
        _topk_kernel,
        out_shape=jax.ShapeDtypeStruct((B, K), jnp.int32),
    )(w_mean.reshape(B, S))

    out_sc = _gather_call(x, idx)

    idx3 = idx.reshape(B, 1, K)
    out_tc = pl.pallas_call(
        _gather_tc,
        grid=(_BTC,),
        in_specs=[
            pl.BlockSpec((1, D, S), lambda b: (b, 0, 0)),
            pl.BlockSpec((1, 1, K), lambda b: (b, 0, 0)),
        ],
        out_specs=pl.BlockSpec((1, D, K), lambda b: (b, 0, 0)),
        out_shape=jax.ShapeDtypeStruct((_BTC, D, K), jnp.float32),
        compiler_params=pltpu.CompilerParams(
            dimension_semantics=("arbitrary",)),
    )(x, idx3)
    return jnp.concatenate([out_tc, out_sc], axis=0)


# speedup vs baseline: 1.6854x; 1.3177x over previous
import functools

import jax
import jax.numpy as jnp
from jax import lax
from jax.experimental import pallas as pl
from jax.experimental.pallas import tpu as pltpu
from jax.experimental.pallas import tpu_sc as plsc

B, D, S = 4, 1024, 2048
K = 256
_ROWS_PER_STEP = 1024
_G, _L = 16, 128

_BTC = 4
_BSC = B - _BTC
_NC, _NS, _SL = 2, 16, 16
_NW = _NC * _NS
_WPB = _NW // max(_BSC, 1)
_ROWS_PER_W = D // _WPB
_CHUNK = 32
_NCHUNK = _ROWS_PER_W // _CHUNK


def _mean_kernel(w_ref, out_ref, acc_ref):
    j = pl.program_id(1)
    nj = pl.num_programs(1)

    @pl.when(j == 0)
    def _init():
        acc_ref[...] = jnp.zeros_like(acc_ref)

    acc = acc_ref[...]
    for g in range(_ROWS_PER_STEP // 8):
        acc = acc + w_ref[0, 8 * g:8 * g + 8, :]
    acc_ref[...] = acc

    @pl.when(j == nj - 1)
    def _finish():
        a = acc_ref[...]
        t = a[0:4] + a[4:8]
        u = t[0:2] + t[2:4]
        s = u[0:1] + u[1:2]
        out_ref[0] = s * (1.0 / S)


def _topk_kernel(m_ref, idx_ref):
    v = m_ref[...].reshape(B, _G, _L)
    jg = lax.broadcasted_iota(jnp.int32, (B, _G, _L), 1)
    jl = lax.broadcasted_iota(jnp.int32, (B, _G, _L), 2)
    i = jg * _L + jl

    def cmpex(v, i, d, k):
        if d >= _L:
            dd = d // _L
            is_lo = (jg & dd) == 0
            vp = jnp.where(is_lo, jnp.roll(v, -dd, axis=1),
                           jnp.roll(v, dd, axis=1))
            ip = jnp.where(is_lo, jnp.roll(i, -dd, axis=1),
                           jnp.roll(i, dd, axis=1))
        else:
            is_lo = (jl & d) == 0
            vp = jnp.where(is_lo, jnp.roll(v, -d, axis=2),
                           jnp.roll(v, d, axis=2))
            ip = jnp.where(is_lo, jnp.roll(i, -d, axis=2),
                           jnp.roll(i, d, axis=2))
        if k >= S:
            asc = jnp.full(v.shape, True)
        elif k >= _L:
            asc = (jg & (k // _L)) == 0
        else:
            asc = (jl & k) == 0
        own_first = (v > vp) | ((v == vp) & (i < ip))
        take_own = own_first == (is_lo == asc)
        return jnp.where(take_own, v, vp), jnp.where(take_own, i, ip)

    k = 2
    while k <= S:
        d = k // 2
        while d >= 1:
            v, i = cmpex(v, i, d, k)
            d //= 2
        k *= 2

    idx_ref[...] = i[:, 0:K // _L, :].reshape(B, K)


def _gather_sc(x_hbm, idx_hbm, out_hbm, idx_v, row_v, orow_v, sem):
    wid = lax.axis_index("s") * _NC + lax.axis_index("c")
    bo = wid // _WPB
    b = bo + _BTC
    r0 = (wid % _WPB) * _ROWS_PER_W
    pltpu.sync_copy(idx_hbm.at[b], idx_v)

    def chunk_body(c, carry):
        base = r0 + c * _CHUNK
        pltpu.sync_copy(x_hbm.at[b, pl.ds(base, _CHUNK)], row_v)
        for i in range(_CHUNK):
            ri = jnp.full((_SL,), i, jnp.int32)
            for j in range(K // _SL):
                ids = idx_v[pl.ds(j * _SL, _SL)]
                orow_v[i, pl.ds(j * _SL, _SL)] = plsc.load_gather(
                    row_v, [ri, ids])
        pltpu.sync_copy(orow_v, out_hbm.at[bo, pl.ds(base, _CHUNK)])
        return carry

    lax.fori_loop(0, _NCHUNK, chunk_body, 0)


_gather_call = pl.kernel(
    _gather_sc,
    mesh=plsc.VectorSubcoreMesh(core_axis_name="c", subcore_axis_name="s"),
    out_type=jax.ShapeDtypeStruct((_BSC, D, K), jnp.float32),
    scratch_types=[
        pltpu.VMEM((K,), jnp.int32),
        pltpu.VMEM((_CHUNK, S), jnp.float32),
        pltpu.VMEM((_CHUNK, K), jnp.float32),
        pltpu.SemaphoreType.DMA,
    ],
    compiler_params=pltpu.CompilerParams(needs_layout_passes=False),
) if _BSC else None


def _gather_tc(x_ref, idx_ref, out_ref):
    idx_row = idx_ref[0]
    onehot = (lax.broadcasted_iota(jnp.int32, (S, K), 0)
              == idx_row).astype(jnp.bfloat16)
    xb = x_ref[0]
    mask = jnp.uint32(0xFFFF0000)
    hi_f = lax.bitcast_convert_type(
        lax.bitcast_convert_type(xb, jnp.uint32) & mask, jnp.float32)
    r = xb - hi_f
    mid_f = lax.bitcast_convert_type(
        lax.bitcast_convert_type(r, jnp.uint32) & mask, jnp.float32)
    lo_f = r - mid_f
    acc = jnp.dot(hi_f.astype(jnp.bfloat16), onehot,
                  preferred_element_type=jnp.float32)
    acc = acc + jnp.dot(mid_f.astype(jnp.bfloat16), onehot,
                        preferred_element_type=jnp.float32)
    acc = acc + jnp.dot(lo_f.astype(jnp.bfloat16), onehot,
                        preferred_element_type=jnp.float32)
    out_ref[0] = acc


@functools.partial(jax.jit)
def kernel(x, w):
    nsteps = S // _ROWS_PER_STEP
    w_mean = pl.pallas_call(
        _mean_kernel,
        grid=(B, nsteps),
        in_specs=[pl.BlockSpec((1, _ROWS_PER_STEP, S),
                               lambda b, j: (b, j, 0))],
        out_specs=pl.BlockSpec((1, 1, S), lambda b, j: (b, 0, 0)),
        out_shape=jax.ShapeDtypeStruct((B, 1, S), jnp.float32),
        scratch_shapes=[pltpu.VMEM((8, S), jnp.float32)],
        compiler_params=pltpu.CompilerParams(
            dimension_semantics=("arbitrary", "arbitrary")),
    )(w)

    idx = pl.pallas_call(
        _topk_kernel,
        out_shape=jax.ShapeDtypeStruct((B, K), jnp.int32),
    )(w_mean.reshape(B, S))

    out_sc = _gather_call(x, idx) if _BSC else None

    idx3 = idx.reshape(B, 1, K)
    out_tc = pl.pallas_call(
        _gather_tc,
        grid=(_BTC,),
        in_specs=[
            pl.BlockSpec((1, D, S), lambda b: (b, 0, 0)),
            pl.BlockSpec((1, 1, K), lambda b: (b, 0, 0)),
        ],
        out_specs=pl.BlockSpec((1, D, K), lambda b: (b, 0, 0)),
        out_shape=jax.ShapeDtypeStruct((_BTC, D, K), jnp.float32),
        compiler_params=pltpu.CompilerParams(
            dimension_semantics=("arbitrary",)),
    )(x, idx3)
    if out_sc is None:
        return out_tc
    return jnp.concatenate([out_tc, out_sc], axis=0)
